# baseline (speedup 1.0000x reference)
import jax
import jax.numpy as jnp
from jax import lax
from jax.experimental import pallas as pl
from jax.experimental.pallas import tpu as pltpu
from jax.experimental.pallas import tpu_sc as plsc

T = 4
B = 128
VOCAB = 100000
ATTN = 200
OOV = 100
VEXT = VOCAB + OOV
ROWS = T * B
LANES = 16
NW = 32
ROWS_PER_W = ROWS // NW
BUF = 100112

IW = 112
SPLIT = 104

VB = 2048
NVB = (VEXT + VB - 1) // VB


def _tc_body(pg_ref, vd_ref, out_ref):
    j = pl.program_id(0)
    v = j * VB + lax.broadcasted_iota(jnp.int32, (VB, T, 128), 0)
    pg = pg_ref[...][None]
    vals = pg * jnp.transpose(vd_ref[...], (1, 0, 2))
    out_ref[...] = jnp.where(v < VOCAB, vals, 0.0)


def _sc_body(attn_hbm, pg_hbm, idx_hbm, out_ref,
             iv0, iv1, av, ov0, ov1, buf, pgv, sem):
    wid = lax.axis_index("s") * 2 + lax.axis_index("c")
    lanes = lax.iota(jnp.int32, LANES)
    zf = jnp.zeros((LANES,), jnp.float32)
    zi = jnp.zeros((LANES,), jnp.int32)

    pltpu.sync_copy(pg_hbm, pgv.at[pl.ds(0, ROWS)])

    def row_body(r, carry):
        row = wid * ROWS_PER_W + r
        b = lax.rem(row, B)
        pltpu.sync_copy(idx_hbm.at[pl.ds(b * ATTN, SPLIT)],
                        iv0.at[pl.ds(0, SPLIT)])
        pltpu.sync_copy(idx_hbm.at[pl.ds(b * ATTN + SPLIT, ATTN - SPLIT)],
                        iv1.at[pl.ds(0, ATTN - SPLIT)])
        pltpu.sync_copy(attn_hbm.at[pl.ds(row * ATTN, SPLIT)],
                        av.at[pl.ds(0, SPLIT)])
        pltpu.sync_copy(attn_hbm.at[pl.ds(row * ATTN + SPLIT, ATTN - SPLIT)],
                        av.at[pl.ds(IW, ATTN - SPLIT)])

        iv0[pl.ds(96, LANES)] = jnp.where(lanes < SPLIT - 96,
                                          iv0[pl.ds(96, LANES)], zi)
        iv1[pl.ds(96, LANES)] = zi
        av[pl.ds(96, LANES)] = jnp.where(lanes < SPLIT - 96,
                                         av[pl.ds(96, LANES)], zf)
        av[pl.ds(IW + 96, LANES)] = zf

        pgwin = pgv[pl.ds(row, LANES)]
        omg = jnp.ones((LANES,), jnp.float32) - (zf + pgwin[0])

        for ivs in (iv0, iv1):
            for c in range(IW // LANES):
                o = c * LANES
                ivs[pl.ds(o, LANES)] = ivs[pl.ds(o, LANES)] * 512 + row

        pltpu.async_copy(out_ref.at[iv0], ov0, sem).wait()
        pltpu.async_copy(out_ref.at[iv1], ov1, sem).wait()

        for ivs, ovs in ((iv0, ov0), (iv1, ov1)):
            for c in range(IW // LANES):
                o = c * LANES
                vloc = lax.shift_right_logical(ivs[pl.ds(o, LANES)], 9)
                plsc.store_scatter(buf, [vloc], ovs[pl.ds(o, LANES)])
        for s, ivs in enumerate((iv0, iv1)):
            for c in range(IW // LANES):
                o = c * LANES
                vloc = lax.shift_right_logical(ivs[pl.ds(o, LANES)], 9)
                vals = av[pl.ds(s * IW + o, LANES)] * omg
                for lane in range(LANES):
                    plsc.addupdate_scatter(buf, [vloc], vals,
                                           mask=lanes == lane)
        for ivs, ovs in ((iv0, ov0), (iv1, ov1)):
            for c in range(IW // LANES):
                o = c * LANES
                vloc = lax.shift_right_logical(ivs[pl.ds(o, LANES)], 9)
                ovs[pl.ds(o, LANES)] = plsc.load_gather(buf, [vloc])

        pltpu.async_copy(ov0, out_ref.at[iv0], sem).wait()
        pltpu.async_copy(ov1, out_ref.at[iv1], sem).wait()
        return carry

    lax.fori_loop(0, ROWS_PER_W, row_body, 0)


@jax.jit
def _final_dist(vocab_dists, attn_dists, p_gens, enc_batch_extend_vocab):
    vocab_t = jnp.transpose(vocab_dists, (0, 2, 1))
    pg2 = p_gens.reshape(T, B)

    dense = pl.pallas_call(
        _tc_body,
        grid=(NVB,),
        in_specs=[
            pl.BlockSpec((T, B), lambda j: (0, 0)),
            pl.BlockSpec((T, VB, B), lambda j: (0, j, 0)),
        ],
        out_specs=pl.BlockSpec((VB, T, B), lambda j: (j, 0, 0)),
        out_shape=jax.ShapeDtypeStruct((VEXT, T, B), jnp.float32),
        compiler_params=pltpu.CompilerParams(
            dimension_semantics=("arbitrary",)),
    )(pg2, vocab_t)

    attn1 = attn_dists.reshape(ROWS * ATTN)
    pg1 = p_gens.reshape(ROWS)
    idx1 = enc_batch_extend_vocab.reshape(B * ATTN)

    mesh = plsc.VectorSubcoreMesh(core_axis_name="c", subcore_axis_name="s")
    rmw = pl.kernel(
        _sc_body,
        out_type=(),
        mesh=mesh,
        compiler_params=pltpu.CompilerParams(needs_layout_passes=False),
        scratch_types=[
            pltpu.VMEM((IW,), jnp.int32),
            pltpu.VMEM((IW,), jnp.int32),
            pltpu.VMEM((2 * IW,), jnp.float32),
            pltpu.VMEM((IW,), jnp.float32),
            pltpu.VMEM((IW,), jnp.float32),
            pltpu.VMEM((BUF,), jnp.float32),
            pltpu.VMEM((ROWS + LANES,), jnp.float32),
            pltpu.SemaphoreType.DMA,
        ],
    )
    ref = jax.new_ref(dense.reshape(VEXT * ROWS))
    rmw(attn1, pg1, idx1, ref)
    out_t = ref[...].reshape(VEXT, T, B)
    return jnp.transpose(out_t, (1, 2, 0))


def kernel(vocab_dists, attn_dists, p_gens, enc_batch_extend_vocab):
    return _final_dist(vocab_dists, attn_dists, p_gens,
                       enc_batch_extend_vocab)
